# baseline (speedup 1.0000x reference)
import math

import jax
import jax.numpy as jnp
from jax import lax
from jax.experimental import pallas as pl
from jax.experimental.pallas import tpu as pltpu

L = 2048
D = 1024
H = 16
DH = 64
BL = 256
NBLK = L // BL
PAD = 1024
OFFSETS = tuple(2 ** j for j in range(11))
SCALE = 1.0 / math.sqrt(DH)


def _fused_kernel(x_ref, wq_ref, wk_ref, wv_ref, wo_ref,
                  bq_ref, bk_ref, bv_ref, bo_ref, o_ref,
                  q_s, kp_s, vp_s, sv_s):
    f32 = jnp.float32
    pid = pl.program_id(0)

    @pl.when(pid < NBLK)
    def _proj():
        r0 = pid * BL
        xb = x_ref[...]
        q_s[pl.ds(r0, BL), :] = (
            jnp.dot(xb, wq_ref[...], preferred_element_type=f32) + bq_ref[...])
        kp_s[pl.ds(PAD + r0, BL), :] = (
            jnp.dot(xb, wk_ref[...], preferred_element_type=f32) + bk_ref[...])
        vb = jnp.dot(xb, wv_ref[...], preferred_element_type=f32) + bv_ref[...]
        vp_s[pl.ds(PAD + r0, BL), :] = vb

        @pl.when(pid < PAD // BL)
        def _():
            kp_s[pl.ds(pid * BL, BL), :] = jnp.zeros((BL, D), f32)
            vp_s[pl.ds(pid * BL, BL), :] = jnp.zeros((BL, D), f32)

        sv = jnp.sum(vb, axis=0, keepdims=True)

        @pl.when(pid == 0)
        def _():
            sv_s[...] = sv

        @pl.when(pid != 0)
        def _():
            sv_s[...] += sv

    @pl.when(pid >= NBLK)
    def _attn():
        i0 = (pid - NBLK) * BL
        q = q_s[pl.ds(i0, BL), :]

        sel = (lax.broadcasted_iota(jnp.int32, (D, H), 0) // DH
               == lax.broadcasted_iota(jnp.int32, (D, H), 1)).astype(f32)
        selT = (lax.broadcasted_iota(jnp.int32, (H, D), 1) // DH
                == lax.broadcasted_iota(jnp.int32, (H, D), 0)).astype(f32)

        kd = kp_s[pl.ds(i0 + PAD, BL), :]
        vd = vp_s[pl.ds(i0 + PAD, BL), :]
        s = jnp.dot(q * kd, sel, preferred_element_type=f32) * SCALE
        w = jnp.exp(s) - 1.0
        z = w + float(L)
        acc = jnp.dot(w, selT, preferred_element_type=f32) * vd

        for d in OFFSETS:
            if d % 8 == 0:
                ks = kp_s[pl.ds(i0 + PAD - d, BL), :]
                vs = vp_s[pl.ds(i0 + PAD - d, BL), :]
            else:
                kw = kp_s[pl.ds(i0 + PAD - 8, BL + 8), :]
                vw = vp_s[pl.ds(i0 + PAD - 8, BL + 8), :]
                ks = kw[8 - d:8 - d + BL, :]
                vs = vw[8 - d:8 - d + BL, :]
            s = jnp.dot(q * ks, sel, preferred_element_type=f32) * SCALE
            w = jnp.exp(s) - 1.0
            z += w
            acc += jnp.dot(w, selT, preferred_element_type=f32) * vs

        att = (acc + sv_s[...]) / jnp.dot(z, selT, preferred_element_type=f32)
        o_ref[...] = (
            jnp.dot(att.astype(jnp.bfloat16), wo_ref[...],
                    preferred_element_type=f32) + bo_ref[...])


@jax.jit
def kernel(x, Wq, bq, Wk, bk, Wv, bv, Wo, bo):
    x2 = x.reshape(L, D).astype(jnp.bfloat16)
    Wq = Wq.astype(jnp.bfloat16)
    Wk = Wk.astype(jnp.bfloat16)
    Wv = Wv.astype(jnp.bfloat16)
    Wo = Wo.astype(jnp.bfloat16)
    bq2 = bq.reshape(1, D)
    bk2 = bk.reshape(1, D)
    bv2 = bv.reshape(1, D)
    bo2 = bo.reshape(1, D)

    full = lambda shape: pl.BlockSpec(shape, lambda i: (0, 0))

    out = pl.pallas_call(
        _fused_kernel,
        grid=(2 * NBLK,),
        in_specs=[
            pl.BlockSpec((BL, D), lambda i: (jnp.minimum(i, NBLK - 1), 0)),
            full((D, D)), full((D, D)), full((D, D)), full((D, D)),
            full((1, D)), full((1, D)), full((1, D)), full((1, D)),
        ],
        out_specs=pl.BlockSpec((BL, D), lambda i: (jnp.maximum(i - NBLK, 0), 0)),
        out_shape=jax.ShapeDtypeStruct((L, D), jnp.float32),
        scratch_shapes=[
            pltpu.VMEM((L, D), jnp.float32),
            pltpu.VMEM((PAD + L, D), jnp.float32),
            pltpu.VMEM((PAD + L, D), jnp.float32),
            pltpu.VMEM((1, D), jnp.float32),
        ],
        compiler_params=pltpu.CompilerParams(
            dimension_semantics=("arbitrary",)),
    )(x2, Wq, Wk, Wv, Wo, bq2, bk2, bv2, bo2)

    return out.reshape(1, L, D)


# speedup vs baseline: 1.0193x; 1.0193x over previous
import math

import jax
import jax.numpy as jnp
from jax import lax
from jax.experimental import pallas as pl
from jax.experimental.pallas import tpu as pltpu

L = 2048
D = 1024
H = 16
DH = 64
BL = 256
NBLK = L // BL
PAD = 1024
OFFSETS = tuple(2 ** j for j in range(11))
SCALE = 1.0 / math.sqrt(DH)


def _fused_kernel(x_ref, wq_ref, wk_ref, wv_ref, wo_ref,
                  bq_ref, bk_ref, bv_ref, bo_ref, o_ref,
                  q_s, kp_s, vp_s, sv_s):
    f32 = jnp.float32
    pid = pl.program_id(0)

    @pl.when(pid < NBLK)
    def _proj():
        r0 = pid * BL
        xb = x_ref[...]
        q_s[pl.ds(r0, BL), :] = (
            jnp.dot(xb, wq_ref[...], preferred_element_type=f32) + bq_ref[...])
        kp_s[pl.ds(PAD + r0, BL), :] = (
            jnp.dot(xb, wk_ref[...], preferred_element_type=f32) + bk_ref[...])
        vb = jnp.dot(xb, wv_ref[...], preferred_element_type=f32) + bv_ref[...]
        vp_s[pl.ds(PAD + r0, BL), :] = vb

        @pl.when(pid < PAD // BL)
        def _():
            kp_s[pl.ds(pid * BL, BL), :] = jnp.zeros((BL, D), f32)
            vp_s[pl.ds(pid * BL, BL), :] = jnp.zeros((BL, D), f32)

        sv = jnp.sum(vb, axis=0, keepdims=True)

        @pl.when(pid == 0)
        def _():
            sv_s[...] = sv

        @pl.when(pid != 0)
        def _():
            sv_s[...] += sv

    @pl.when(pid >= NBLK)
    def _attn():
        i0 = (pid - NBLK) * BL
        q = q_s[pl.ds(i0, BL), :]

        bf16 = jnp.bfloat16
        sel = (lax.broadcasted_iota(jnp.int32, (D, H), 0) // DH
               == lax.broadcasted_iota(jnp.int32, (D, H), 1)).astype(bf16)
        selT = (lax.broadcasted_iota(jnp.int32, (H, D), 1) // DH
                == lax.broadcasted_iota(jnp.int32, (H, D), 0)).astype(bf16)

        kw = kp_s[pl.ds(i0 + PAD - 8, BL + 8), :]
        vw = vp_s[pl.ds(i0 + PAD - 8, BL + 8), :]

        kd = kw[8:8 + BL, :]
        vd = vw[8:8 + BL, :]
        s = jnp.dot((q * kd).astype(bf16), sel,
                    preferred_element_type=f32) * SCALE
        w = jnp.exp(s) - 1.0
        z = w + float(L)
        acc = jnp.dot(w.astype(bf16), selT, preferred_element_type=f32) * vd

        for d in OFFSETS:
            if d % 8 == 0:
                ks = kp_s[pl.ds(i0 + PAD - d, BL), :]
                vs = vp_s[pl.ds(i0 + PAD - d, BL), :]
            else:
                ks = kw[8 - d:8 - d + BL, :]
                vs = vw[8 - d:8 - d + BL, :]
            s = jnp.dot((q * ks).astype(bf16), sel,
                        preferred_element_type=f32) * SCALE
            w = jnp.exp(s) - 1.0
            z += w
            acc += jnp.dot(w.astype(bf16), selT,
                           preferred_element_type=f32) * vs

        selT32 = selT.astype(f32)
        att = (acc + sv_s[...]) / jnp.dot(z, selT32, preferred_element_type=f32)
        o_ref[...] = (
            jnp.dot(att, wo_ref[...], preferred_element_type=f32) + bo_ref[...])


@jax.jit
def kernel(x, Wq, bq, Wk, bk, Wv, bv, Wo, bo):
    x2 = x.reshape(L, D)
    bq2 = bq.reshape(1, D)
    bk2 = bk.reshape(1, D)
    bv2 = bv.reshape(1, D)
    bo2 = bo.reshape(1, D)

    full = lambda shape: pl.BlockSpec(shape, lambda i: (0, 0))

    out = pl.pallas_call(
        _fused_kernel,
        grid=(2 * NBLK,),
        in_specs=[
            pl.BlockSpec((BL, D), lambda i: (jnp.minimum(i, NBLK - 1), 0)),
            full((D, D)), full((D, D)), full((D, D)), full((D, D)),
            full((1, D)), full((1, D)), full((1, D)), full((1, D)),
        ],
        out_specs=pl.BlockSpec((BL, D), lambda i: (jnp.maximum(i - NBLK, 0), 0)),
        out_shape=jax.ShapeDtypeStruct((L, D), jnp.float32),
        scratch_shapes=[
            pltpu.VMEM((L, D), jnp.float32),
            pltpu.VMEM((PAD + L, D), jnp.float32),
            pltpu.VMEM((PAD + L, D), jnp.float32),
            pltpu.VMEM((1, D), jnp.float32),
        ],
        compiler_params=pltpu.CompilerParams(
            dimension_semantics=("arbitrary",)),
    )(x2, Wq, Wk, Wv, Wo, bq2, bk2, bv2, bo2)

    return out.reshape(1, L, D)


# speedup vs baseline: 1.1728x; 1.1507x over previous
import math

import jax
import jax.numpy as jnp
from jax import lax
from jax.experimental import pallas as pl
from jax.experimental.pallas import tpu as pltpu

L = 2048
D = 1024
H = 16
DH = 64
BL = 256
NBLK = L // BL
PAD = 1024
OFFSETS = tuple(2 ** j for j in range(11))
SCALE = 1.0 / math.sqrt(DH)


def _fused_kernel(x_ref, wq_ref, wk_ref, wv_ref, wo_ref,
                  bq_ref, bk_ref, bv_ref, bo_ref, o_ref,
                  q_s, kp_s, vp_s, sv_s):
    f32 = jnp.float32
    pid = pl.program_id(0)

    @pl.when(pid < NBLK)
    def _proj():
        r0 = pid * BL
        xb = x_ref[...]
        q_s[pl.ds(r0, BL), :] = (
            jnp.dot(xb, wq_ref[...], preferred_element_type=f32) + bq_ref[...])
        kp_s[pl.ds(PAD + r0, BL), :] = (
            jnp.dot(xb, wk_ref[...], preferred_element_type=f32) + bk_ref[...])
        vb = jnp.dot(xb, wv_ref[...], preferred_element_type=f32) + bv_ref[...]
        vp_s[pl.ds(PAD + r0, BL), :] = vb

        @pl.when(pid < PAD // BL)
        def _():
            kp_s[pl.ds(pid * BL, BL), :] = jnp.zeros((BL, D), f32)
            vp_s[pl.ds(pid * BL, BL), :] = jnp.zeros((BL, D), f32)

        sv = jnp.sum(vb, axis=0, keepdims=True)

        @pl.when(pid == 0)
        def _():
            sv_s[...] = sv

        @pl.when(pid != 0)
        def _():
            sv_s[...] += sv

    @pl.when(pid >= NBLK)
    def _attn():
        i0 = (pid - NBLK) * BL
        q = q_s[pl.ds(i0, BL), :]

        sel = (lax.broadcasted_iota(jnp.int32, (D, H), 0) // DH
               == lax.broadcasted_iota(jnp.int32, (D, H), 1)).astype(f32)
        selT = (lax.broadcasted_iota(jnp.int32, (H, D), 1) // DH
                == lax.broadcasted_iota(jnp.int32, (H, D), 0)).astype(f32)

        kw = kp_s[pl.ds(i0 + PAD - 8, BL + 8), :]
        vw = vp_s[pl.ds(i0 + PAD - 8, BL + 8), :]

        kd = kw[8:8 + BL, :]
        vd = vw[8:8 + BL, :]
        s = jnp.dot(q * kd, sel, preferred_element_type=f32) * SCALE
        w = jnp.exp(s) - 1.0
        z = w + float(L)
        acc = jnp.dot(w, selT, preferred_element_type=f32) * vd

        for d in OFFSETS:
            if d % 8 == 0:
                ks = kp_s[pl.ds(i0 + PAD - d, BL), :]
                vs = vp_s[pl.ds(i0 + PAD - d, BL), :]
            else:
                ks = kw[8 - d:8 - d + BL, :]
                vs = vw[8 - d:8 - d + BL, :]
            s = jnp.dot(q * ks, sel, preferred_element_type=f32) * SCALE
            w = jnp.exp(s) - 1.0
            z += w
            acc += jnp.dot(w, selT, preferred_element_type=f32) * vs

        att = (acc + sv_s[...]) / jnp.dot(z, selT, preferred_element_type=f32)
        o_ref[...] = (
            jnp.dot(att, wo_ref[...], preferred_element_type=f32) + bo_ref[...])


@jax.jit
def kernel(x, Wq, bq, Wk, bk, Wv, bv, Wo, bo):
    x2 = x.reshape(L, D)
    bq2 = bq.reshape(1, D)
    bk2 = bk.reshape(1, D)
    bv2 = bv.reshape(1, D)
    bo2 = bo.reshape(1, D)

    full = lambda shape: pl.BlockSpec(shape, lambda i: (0, 0))

    out = pl.pallas_call(
        _fused_kernel,
        grid=(2 * NBLK,),
        in_specs=[
            pl.BlockSpec((BL, D), lambda i: (jnp.minimum(i, NBLK - 1), 0)),
            full((D, D)), full((D, D)), full((D, D)), full((D, D)),
            full((1, D)), full((1, D)), full((1, D)), full((1, D)),
        ],
        out_specs=pl.BlockSpec((BL, D), lambda i: (jnp.maximum(i - NBLK, 0), 0)),
        out_shape=jax.ShapeDtypeStruct((L, D), jnp.float32),
        scratch_shapes=[
            pltpu.VMEM((L, D), jnp.float32),
            pltpu.VMEM((PAD + L, D), jnp.float32),
            pltpu.VMEM((PAD + L, D), jnp.float32),
            pltpu.VMEM((1, D), jnp.float32),
        ],
        compiler_params=pltpu.CompilerParams(
            dimension_semantics=("arbitrary",)),
    )(x2, Wq, Wk, Wv, Wo, bq2, bk2, bv2, bo2)

    return out.reshape(1, L, D)


# speedup vs baseline: 1.2194x; 1.0397x over previous
import math

import jax
import jax.numpy as jnp
from jax import lax
from jax.experimental import pallas as pl
from jax.experimental.pallas import tpu as pltpu

L = 2048
D = 1024
H = 16
DH = 64
BL = 512
NBLK = L // BL
PAD = 1024
OFFSETS = tuple(2 ** j for j in range(11))
SCALE = 1.0 / math.sqrt(DH)


def _fused_kernel(x_ref, wq_ref, wk_ref, wv_ref, wo_ref,
                  bq_ref, bk_ref, bv_ref, bo_ref, o_ref,
                  q_s, kp_s, vp_s, sv_s):
    f32 = jnp.float32
    pid = pl.program_id(0)

    @pl.when(pid < NBLK)
    def _proj():
        r0 = pid * BL
        xb = x_ref[...]
        q_s[pl.ds(r0, BL), :] = (
            jnp.dot(xb, wq_ref[...], preferred_element_type=f32) + bq_ref[...])
        kp_s[pl.ds(PAD + r0, BL), :] = (
            jnp.dot(xb, wk_ref[...], preferred_element_type=f32) + bk_ref[...])
        vb = jnp.dot(xb, wv_ref[...], preferred_element_type=f32) + bv_ref[...]
        vp_s[pl.ds(PAD + r0, BL), :] = vb

        @pl.when(pid < PAD // BL)
        def _():
            kp_s[pl.ds(pid * BL, BL), :] = jnp.zeros((BL, D), f32)
            vp_s[pl.ds(pid * BL, BL), :] = jnp.zeros((BL, D), f32)

        sv = jnp.sum(vb, axis=0, keepdims=True)

        @pl.when(pid == 0)
        def _():
            sv_s[...] = sv

        @pl.when(pid != 0)
        def _():
            sv_s[...] += sv

    @pl.when(pid >= NBLK)
    def _attn():
        i0 = (pid - NBLK) * BL
        q = q_s[pl.ds(i0, BL), :]

        sel = (lax.broadcasted_iota(jnp.int32, (D, H), 0) // DH
               == lax.broadcasted_iota(jnp.int32, (D, H), 1)).astype(f32)
        selT = (lax.broadcasted_iota(jnp.int32, (H, D), 1) // DH
                == lax.broadcasted_iota(jnp.int32, (H, D), 0)).astype(f32)

        kw = kp_s[pl.ds(i0 + PAD - 8, BL + 8), :]
        vw = vp_s[pl.ds(i0 + PAD - 8, BL + 8), :]

        kd = kw[8:8 + BL, :]
        vd = vw[8:8 + BL, :]
        s = jnp.dot(q * kd, sel, preferred_element_type=f32) * SCALE
        w = jnp.exp(s) - 1.0
        z = w + float(L)
        acc = jnp.dot(w, selT, preferred_element_type=f32) * vd

        for d in OFFSETS:
            if d % 8 == 0:
                ks = kp_s[pl.ds(i0 + PAD - d, BL), :]
                vs = vp_s[pl.ds(i0 + PAD - d, BL), :]
            else:
                ks = kw[8 - d:8 - d + BL, :]
                vs = vw[8 - d:8 - d + BL, :]
            s = jnp.dot(q * ks, sel, preferred_element_type=f32) * SCALE
            w = jnp.exp(s) - 1.0
            z += w
            acc += jnp.dot(w, selT, preferred_element_type=f32) * vs

        att = (acc + sv_s[...]) / jnp.dot(z, selT, preferred_element_type=f32)
        o_ref[...] = (
            jnp.dot(att, wo_ref[...], preferred_element_type=f32) + bo_ref[...])


@jax.jit
def kernel(x, Wq, bq, Wk, bk, Wv, bv, Wo, bo):
    x2 = x.reshape(L, D)
    bq2 = bq.reshape(1, D)
    bk2 = bk.reshape(1, D)
    bv2 = bv.reshape(1, D)
    bo2 = bo.reshape(1, D)

    full = lambda shape: pl.BlockSpec(shape, lambda i: (0, 0))

    out = pl.pallas_call(
        _fused_kernel,
        grid=(2 * NBLK,),
        in_specs=[
            pl.BlockSpec((BL, D), lambda i: (jnp.minimum(i, NBLK - 1), 0)),
            full((D, D)), full((D, D)), full((D, D)), full((D, D)),
            full((1, D)), full((1, D)), full((1, D)), full((1, D)),
        ],
        out_specs=pl.BlockSpec((BL, D), lambda i: (jnp.maximum(i - NBLK, 0), 0)),
        out_shape=jax.ShapeDtypeStruct((L, D), jnp.float32),
        scratch_shapes=[
            pltpu.VMEM((L, D), jnp.float32),
            pltpu.VMEM((PAD + L, D), jnp.float32),
            pltpu.VMEM((PAD + L, D), jnp.float32),
            pltpu.VMEM((1, D), jnp.float32),
        ],
        compiler_params=pltpu.CompilerParams(
            dimension_semantics=("arbitrary",),
            vmem_limit_bytes=100 * 1024 * 1024),
    )(x2, Wq, Wk, Wv, Wo, bq2, bk2, bv2, bo2)

    return out.reshape(1, L, D)


# speedup vs baseline: 1.2224x; 1.0025x over previous
import math

import jax
import jax.numpy as jnp
from jax import lax
from jax.experimental import pallas as pl
from jax.experimental.pallas import tpu as pltpu

L = 2048
D = 1024
H = 16
DH = 64
BL = 512
NBLK = L // BL
PAD = 1024
OFFSETS = tuple(2 ** j for j in range(11))
SCALE = 1.0 / math.sqrt(DH)


def _selectors(dtype):
    sel = (lax.broadcasted_iota(jnp.int32, (D, H), 0) // DH
           == lax.broadcasted_iota(jnp.int32, (D, H), 1)).astype(dtype)
    selT = (lax.broadcasted_iota(jnp.int32, (H, D), 1) // DH
            == lax.broadcasted_iota(jnp.int32, (H, D), 0)).astype(dtype)
    return sel, selT


def _fused_kernel(x_ref, wq_ref, wk_ref, wv_ref, wo_ref,
                  bq_ref, bk_ref, bv_ref, bo_ref, o_ref,
                  kp_s, vp_s, u_s):
    f32 = jnp.float32
    pid = pl.program_id(0)

    @pl.when(pid == 0)
    def _vproj():
        vb = (jnp.dot(x_ref[...], wv_ref[...], preferred_element_type=f32)
              + bv_ref[...])
        vp_s[pl.ds(PAD, L), :] = vb
        kp_s[pl.ds(0, PAD), :] = jnp.zeros((PAD, D), f32)
        vp_s[pl.ds(0, PAD), :] = jnp.zeros((PAD, D), f32)
        sv = jnp.sum(vb, axis=0, keepdims=True)
        _, selT = _selectors(f32)
        u_s[...] = jnp.dot(selT * sv, wo_ref[...], preferred_element_type=f32)

    @pl.when(pid > 0)
    def _block():
        b = pid - 1
        i0 = b * BL
        xb = x_ref[pl.ds(i0, BL), :]
        q = (jnp.dot(xb, wq_ref[...], preferred_element_type=f32)
             + bq_ref[...])
        kb = (jnp.dot(xb, wk_ref[...], preferred_element_type=f32)
              + bk_ref[...])
        kp_s[pl.ds(PAD + i0, BL), :] = kb

        sel, selT = _selectors(f32)

        kw = kp_s[pl.ds(i0 + PAD - 8, BL + 8), :]
        vw = vp_s[pl.ds(i0 + PAD - 8, BL + 8), :]

        kd = kw[8:8 + BL, :]
        vd = vw[8:8 + BL, :]
        s = jnp.dot(q * kd, sel, preferred_element_type=f32) * SCALE
        w = jnp.exp(s) - 1.0
        z = w + float(L)
        acc = jnp.dot(w, selT, preferred_element_type=f32) * vd

        for d in OFFSETS:
            if d % 8 == 0:
                ks = kp_s[pl.ds(i0 + PAD - d, BL), :]
                vs = vp_s[pl.ds(i0 + PAD - d, BL), :]
            else:
                ks = kw[8 - d:8 - d + BL, :]
                vs = vw[8 - d:8 - d + BL, :]
            s = jnp.dot(q * ks, sel, preferred_element_type=f32) * SCALE
            w = jnp.exp(s) - 1.0
            z += w
            acc += jnp.dot(w, selT, preferred_element_type=f32) * vs

        zinv = 1.0 / z
        att_main = acc * jnp.dot(zinv, selT, preferred_element_type=f32)
        o_ref[...] = (
            jnp.dot(att_main, wo_ref[...], preferred_element_type=f32)
            + jnp.dot(zinv, u_s[...], preferred_element_type=f32)
            + bo_ref[...])


@jax.jit
def kernel(x, Wq, bq, Wk, bk, Wv, bv, Wo, bo):
    x2 = x.reshape(L, D)
    bq2 = bq.reshape(1, D)
    bk2 = bk.reshape(1, D)
    bv2 = bv.reshape(1, D)
    bo2 = bo.reshape(1, D)

    full = lambda shape: pl.BlockSpec(shape, lambda i: (0, 0))

    out = pl.pallas_call(
        _fused_kernel,
        grid=(1 + NBLK,),
        in_specs=[
            full((L, D)),
            full((D, D)), full((D, D)), full((D, D)), full((D, D)),
            full((1, D)), full((1, D)), full((1, D)), full((1, D)),
        ],
        out_specs=pl.BlockSpec((BL, D), lambda i: (jnp.maximum(i - 1, 0), 0)),
        out_shape=jax.ShapeDtypeStruct((L, D), jnp.float32),
        scratch_shapes=[
            pltpu.VMEM((PAD + L, D), jnp.float32),
            pltpu.VMEM((PAD + L, D), jnp.float32),
            pltpu.VMEM((H, D), jnp.float32),
        ],
        compiler_params=pltpu.CompilerParams(
            dimension_semantics=("arbitrary",),
            vmem_limit_bytes=100 * 1024 * 1024),
    )(x2, Wq, Wk, Wv, Wo, bq2, bk2, bv2, bo2)

    return out.reshape(1, L, D)
